# baseline (speedup 1.0000x reference)
import jax
import jax.numpy as jnp
from jax import lax
from jax.experimental import pallas as pl
from jax.experimental.pallas import tpu as pltpu
from jax.experimental.pallas import tpu_sc as plsc

_PH = 7
_PW = 7
_STRIDE = 16.0
_K = 4
_NW = 32
_CHUNKS = 16


def _bin_bounds(lo, hi, nbins, limit):
    size = jnp.maximum(hi - lo + 1, 1).astype(jnp.float32)
    bs = size / float(nbins)
    p = jnp.arange(nbins, dtype=jnp.float32)
    start = jnp.floor(p[None, :] * bs[:, None]).astype(jnp.int32) + lo[:, None]
    end = jnp.ceil((p[None, :] + 1.0) * bs[:, None]).astype(jnp.int32) + lo[:, None]
    return jnp.clip(start, 0, limit), jnp.clip(end, 0, limit)


def _cell_indices(features_shape, rois):
    B, C, H, W = features_shape
    N = rois.shape[0]
    zero_row = B * H * W

    b_idx = rois[:, 0].astype(jnp.int32)
    coords = jnp.round(rois[:, 1:] * (1.0 / _STRIDE)).astype(jnp.int32)
    x1, y1, x2, y2 = coords[:, 0], coords[:, 1], coords[:, 2], coords[:, 3]
    hs, he = _bin_bounds(y1, y2, _PH, H)
    ws, we = _bin_bounds(x1, x2, _PW, W)

    kh = jnp.arange(16, dtype=jnp.int32) // _K
    kw = jnp.arange(16, dtype=jnp.int32) % _K
    hc = hs[:, :, None] + kh[None, None, :]
    wc = ws[:, :, None] + kw[None, None, :]
    vh = hc < he[:, :, None]
    vw = wc < we[:, :, None]
    hc4 = jnp.minimum(hc, H - 1)[:, :, None, :]
    wc4 = jnp.minimum(wc, W - 1)[:, None, :, :]
    valid = vh[:, :, None, :] & vw[:, None, :, :]
    rowid = (b_idx[:, None, None, None] * H + hc4) * W + wc4
    fv = ((b_idx[:, None, None] * H + jnp.minimum(hs, H - 1)[:, :, None]) * W
          + jnp.minimum(ws, W - 1)[:, None, :])
    idx = jnp.where(valid, rowid, fv[:, :, :, None])
    empty = (he <= hs)[:, :, None] | (we <= ws)[:, None, :]
    idx = jnp.where(empty[:, :, :, None], zero_row, idx)
    return idx.reshape(N, _PH * _PW, 16).astype(jnp.int32)


def kernel(features, rois):
    B, C, H, W = features.shape
    N = rois.shape[0]
    nbins = _PH * _PW

    table = jnp.transpose(features, (0, 2, 3, 1)).reshape(B * H * W, C)
    table = jnp.pad(table, ((0, 8), (0, 0)))
    cellidx = _cell_indices(features.shape, rois)

    mesh = plsc.VectorSubcoreMesh(
        core_axis_name="c", subcore_axis_name="s", num_cores=2, num_subcores=16)

    import functools

    @functools.partial(
        pl.kernel,
        mesh=mesh,
        out_type=jax.ShapeDtypeStruct((N, nbins, C), jnp.float32),
        scratch_types=[
            pltpu.VMEM((nbins, 16), jnp.int32),
            pltpu.VMEM((16, C), jnp.float32),
            pltpu.VMEM((16, C), jnp.float32),
            pltpu.VMEM((nbins, C), jnp.float32),
            pltpu.SemaphoreType.DMA,
            pltpu.SemaphoreType.DMA,
        ],
    )
    def sc_pool(table_hbm, idx_hbm, out_hbm, idx_v, buf0, buf1, out_v, sem0, sem1):
        wid = lax.axis_index("s") * 2 + lax.axis_index("c")

        def start(bin_i, buf, sem):
            pltpu.make_async_copy(table_hbm.at[idx_v.at[bin_i]], buf, sem).start()

        def wait(buf, sem):
            pltpu.make_async_copy(table_hbm.at[idx_v.at[0]], buf, sem).wait()

        def compute(buf, bin_i):
            for c in range(_CHUNKS):
                sl = pl.ds(c * 16, 16)
                vals = [buf[k, sl] for k in range(16)]
                while len(vals) > 1:
                    vals = [jnp.maximum(vals[i], vals[i + 1])
                            for i in range(0, len(vals) - 1, 2)] + (
                        [vals[-1]] if len(vals) % 2 else [])
                out_v[bin_i, sl] = vals[0]

        def roi_body(t, carry):
            r = wid + _NW * t

            @pl.when(r < N)
            def _():
                pltpu.sync_copy(idx_hbm.at[r], idx_v)

                def bin_body(bi, c2):
                    start(bi, buf0, sem0)
                    wait(buf0, sem0)
                    compute(buf0, bi)
                    return c2

                lax.fori_loop(0, nbins, bin_body, 0)
                pltpu.sync_copy(out_v, out_hbm.at[r])

            return carry

        lax.fori_loop(0, (N + _NW - 1) // _NW, roi_body, 0)

    out = sc_pool(table, cellidx)
    return jnp.transpose(out.reshape(N, _PH, _PW, C), (0, 3, 1, 2))


# speedup vs baseline: 1.6633x; 1.6633x over previous
import jax
import jax.numpy as jnp
from jax import lax
from jax.experimental import pallas as pl
from jax.experimental.pallas import tpu as pltpu
from jax.experimental.pallas import tpu_sc as plsc

_PH = 7
_PW = 7
_STRIDE = 16.0
_K = 4
_NW = 32
_CHUNKS = 16


def _bin_bounds(lo, hi, nbins, limit):
    size = jnp.maximum(hi - lo + 1, 1).astype(jnp.float32)
    bs = size / float(nbins)
    p = jnp.arange(nbins, dtype=jnp.float32)
    start = jnp.floor(p[None, :] * bs[:, None]).astype(jnp.int32) + lo[:, None]
    end = jnp.ceil((p[None, :] + 1.0) * bs[:, None]).astype(jnp.int32) + lo[:, None]
    return jnp.clip(start, 0, limit), jnp.clip(end, 0, limit)


def _cell_indices(features_shape, rois):
    B, C, H, W = features_shape
    N = rois.shape[0]
    zero_row = B * H * W

    b_idx = rois[:, 0].astype(jnp.int32)
    coords = jnp.round(rois[:, 1:] * (1.0 / _STRIDE)).astype(jnp.int32)
    x1, y1, x2, y2 = coords[:, 0], coords[:, 1], coords[:, 2], coords[:, 3]
    hs, he = _bin_bounds(y1, y2, _PH, H)
    ws, we = _bin_bounds(x1, x2, _PW, W)

    kh = jnp.arange(16, dtype=jnp.int32) // _K
    kw = jnp.arange(16, dtype=jnp.int32) % _K
    hc = hs[:, :, None] + kh[None, None, :]
    wc = ws[:, :, None] + kw[None, None, :]
    vh = hc < he[:, :, None]
    vw = wc < we[:, :, None]
    hc4 = jnp.minimum(hc, H - 1)[:, :, None, :]
    wc4 = jnp.minimum(wc, W - 1)[:, None, :, :]
    valid = vh[:, :, None, :] & vw[:, None, :, :]
    rowid = (b_idx[:, None, None, None] * H + hc4) * W + wc4
    fv = ((b_idx[:, None, None] * H + jnp.minimum(hs, H - 1)[:, :, None]) * W
          + jnp.minimum(ws, W - 1)[:, None, :])
    idx = jnp.where(valid, rowid, fv[:, :, :, None])
    empty = (he <= hs)[:, :, None] | (we <= ws)[:, None, :]
    idx = jnp.where(empty[:, :, :, None], zero_row, idx)
    return idx.reshape(N, _PH * _PW, 16).astype(jnp.int32)


def kernel(features, rois):
    B, C, H, W = features.shape
    N = rois.shape[0]
    nbins = _PH * _PW

    table = jnp.transpose(features, (0, 2, 3, 1)).reshape(B * H * W, C)
    table = jnp.pad(table, ((0, 8), (0, 0)))
    cellidx = _cell_indices(features.shape, rois)

    mesh = plsc.VectorSubcoreMesh(
        core_axis_name="c", subcore_axis_name="s", num_cores=2, num_subcores=16)

    import functools

    @functools.partial(
        pl.kernel,
        mesh=mesh,
        out_type=jax.ShapeDtypeStruct((N, nbins, C), jnp.float32),
        scratch_types=[
            pltpu.VMEM((nbins, 16), jnp.int32),
            pltpu.VMEM((16, C), jnp.float32),
            pltpu.VMEM((16, C), jnp.float32),
            pltpu.VMEM((nbins, C), jnp.float32),
            pltpu.SemaphoreType.DMA,
            pltpu.SemaphoreType.DMA,
        ],
    )
    def sc_pool(table_hbm, idx_hbm, out_hbm, idx_v, buf0, buf1, out_v, sem0, sem1):
        wid = lax.axis_index("s") * 2 + lax.axis_index("c")

        def start(bin_i, buf, sem):
            pltpu.make_async_copy(table_hbm.at[idx_v.at[bin_i]], buf, sem).start()

        def wait(buf, sem):
            pltpu.make_async_copy(table_hbm.at[idx_v.at[0]], buf, sem).wait()

        def compute(buf, bin_i):
            for c in range(_CHUNKS):
                sl = pl.ds(c * 16, 16)
                vals = [buf[k, sl] for k in range(16)]
                while len(vals) > 1:
                    vals = [jnp.maximum(vals[i], vals[i + 1])
                            for i in range(0, len(vals) - 1, 2)] + (
                        [vals[-1]] if len(vals) % 2 else [])
                out_v[bin_i, sl] = vals[0]

        def roi_body(t, carry):
            r = wid + _NW * t

            @pl.when(r < N)
            def _():
                pltpu.sync_copy(idx_hbm.at[r], idx_v)
                start(0, buf0, sem0)

                def pair_body(p, c2):
                    b0 = 2 * p
                    start(b0 + 1, buf1, sem1)
                    wait(buf0, sem0)
                    compute(buf0, b0)
                    start(b0 + 2, buf0, sem0)
                    wait(buf1, sem1)
                    compute(buf1, b0 + 1)
                    return c2

                lax.fori_loop(0, (nbins - 1) // 2, pair_body, 0)
                wait(buf0, sem0)
                compute(buf0, nbins - 1)
                pltpu.sync_copy(out_v, out_hbm.at[r])

            return carry

        lax.fori_loop(0, (N + _NW - 1) // _NW, roi_body, 0)

    out = sc_pool(table, cellidx)
    return jnp.transpose(out.reshape(N, _PH, _PW, C), (0, 3, 1, 2))
